# Initial kernel scaffold; baseline (speedup 1.0000x reference)
import jax
import jax.numpy as jnp
from jax.experimental import pallas as pl


def kernel(x_feats, gamma_pre, beta_pre, W1, b1, gamma2, beta2, W2, b2, gamma3, beta3, W3, b3, edge_in, edge_out, cu_kernel):
    raise NotImplementedError("write your pallas kernel here")


# speedup vs baseline: 21.0825x; 21.0825x over previous
import functools

import jax
import jax.numpy as jnp
from jax import lax
from jax.experimental import pallas as pl
from jax.experimental.pallas import tpu as pltpu
from jax.experimental.pallas import tpu_sc as plsc

F32 = jnp.float32
EPS = 1e-5
NC = 2
NS = 16
CHUNK = 128
BR = 2000


def _stats_kernel(x_ref, o_ref, acc):
    i = pl.program_id(0)
    nb = pl.num_programs(0)
    xb = x_ref[...]
    upd = jnp.concatenate(
        [jnp.sum(xb, axis=0, keepdims=True),
         jnp.sum(xb * xb, axis=0, keepdims=True)], axis=0)
    acc[...] = jnp.where(i == 0, upd, acc[...] + upd)

    @pl.when(i == nb - 1)
    def _():
        o_ref[...] = acc[...]


def _h1_kernel(x_ref, a_ref, b_ref, w1_ref, b1_ref, h1_ref, s_ref, acc):
    i = pl.program_id(0)
    nb = pl.num_programs(0)
    xp = jnp.maximum(x_ref[...] * a_ref[...] + b_ref[...], 0.0)
    h1 = jnp.dot(xp, w1_ref[...], preferred_element_type=F32) + b1_ref[...]
    h1_ref[...] = h1
    upd = jnp.concatenate(
        [jnp.sum(h1, axis=0, keepdims=True),
         jnp.sum(h1 * h1, axis=0, keepdims=True)], axis=0)
    acc[...] = jnp.where(i == 0, upd, acc[...] + upd)

    @pl.when(i == nb - 1)
    def _():
        s_ref[...] = acc[...]


def _proj_kernel(h1_ref, a_ref, b_ref, w2_ref, p_ref, *, kk):
    h = jnp.maximum(h1_ref[...] * a_ref[...] + b_ref[...], 0.0)
    for k in range(kk):
        p_ref[k, :, :] = jnp.dot(h, w2_ref[k, :, :], preferred_element_type=F32)


def _stats2_kernel(pp_ref, p4_ref, o_ref, acc):
    i = pl.program_id(0)
    nb = pl.num_programs(0)
    s = pp_ref[0, :, :] + pp_ref[1, :, :] + p4_ref[0, :, :]
    upd = jnp.concatenate(
        [jnp.sum(s, axis=0, keepdims=True),
         jnp.sum(s * s, axis=0, keepdims=True)], axis=0)
    acc[...] = jnp.where(i == 0, upd, acc[...] + upd)

    @pl.when(i == nb - 1)
    def _():
        o_ref[...] = acc[...]


def _final_kernel(pp_ref, p4_ref, x_ref, a3_ref, b3f_ref, w3_ref, bias3_ref,
                  ax_ref, bx_ref, y_ref):
    s = pp_ref[0, :, :] + pp_ref[1, :, :] + p4_ref[0, :, :]
    h3 = jnp.maximum(s * a3_ref[...] + b3f_ref[...], 0.0)
    shortcut = jnp.maximum(x_ref[...] * ax_ref[...] + bx_ref[...], 0.0)
    y_ref[...] = (jnp.dot(h3, w3_ref[...], preferred_element_type=F32)
                  + bias3_ref[...] + shortcut)


def _sc_body(p_hbm, g_hbm, o_hbm, out_hbm, gix, oix, rows, zbuf, acc, sem,
             *, ch, acc_rows, n_pad, mid):
    c = lax.axis_index("c")
    s = lax.axis_index("s")

    def zrow(i, carry):
        for j in range(mid // 16):
            zbuf[i, pl.ds(j * 16, 16)] = jnp.zeros((16,), F32)
        return carry
    lax.fori_loop(0, CHUNK, zrow, 0)

    zps = acc_rows // NS
    for r in range(zps // CHUNK):
        pltpu.sync_copy(zbuf, acc.at[pl.ds(s * zps + r * CHUNK, CHUNK)])
    plsc.subcore_barrier()

    pltpu.sync_copy(g_hbm.at[c, s], gix)
    pltpu.sync_copy(o_hbm.at[c, s], oix)

    def body(j, carry):
        pltpu.async_copy(p_hbm.at[gix.at[j]], rows, sem).wait()
        pltpu.sync_copy(rows, acc.at[oix.at[j]], add=True)
        return carry
    lax.fori_loop(0, ch, body, 0)
    plsc.subcore_barrier()

    rps = n_pad // NS
    pltpu.sync_copy(acc.at[pl.ds(s * rps, rps)],
                    out_hbm.at[c, pl.ds(s * rps, rps)])


def kernel(x_feats, gamma_pre, beta_pre, W1, b1, gamma2, beta2, W2, b2,
           gamma3, beta3, W3, b3, edge_in, edge_out, cu_kernel):
    n, cin = x_feats.shape
    mid = W1.shape[1]
    cout = W3.shape[1]
    kk = W2.shape[0]
    e_total = edge_in.shape[0]
    nb = n // BR
    kc = kk // 2

    cu = cu_kernel.astype(jnp.int32)
    e8 = e_total - n
    pos2 = jnp.arange(e8, dtype=jnp.int32)
    src_pos = pos2 + jnp.where(pos2 >= cu[kc], n, 0).astype(jnp.int32)
    ei = edge_in.astype(jnp.int32)[src_pos]
    eo = edge_out.astype(jnp.int32)[src_pos]
    seg = jnp.zeros((e8,), jnp.int32)
    for k in range(1, kk):
        seg = seg + (src_pos >= cu[k]).astype(jnp.int32)
    gidx = ei + seg * n

    nw = NC * NS
    ch = -(-e8 // (nw * CHUNK))
    ep = nw * ch * CHUNK
    pad = ep - e8
    gidx = jnp.concatenate([gidx, jnp.zeros((pad,), jnp.int32)])
    oidx = jnp.concatenate([eo, jnp.full((pad,), n, jnp.int32)])
    g4 = gidx.reshape(NC, NS, ch, CHUNK)
    o4 = oidx.reshape(NC, NS, ch, CHUNK)

    sums_x = pl.pallas_call(
        _stats_kernel, grid=(nb,),
        in_specs=[pl.BlockSpec((BR, cin), lambda i: (i, 0))],
        out_specs=pl.BlockSpec((2, cin), lambda i: (0, 0)),
        out_shape=jax.ShapeDtypeStruct((2, cin), F32),
        scratch_shapes=[pltpu.VMEM((2, cin), F32)],
    )(x_feats)
    mx = sums_x[0] / n
    ax = gamma_pre / jnp.sqrt(sums_x[1] / n - mx * mx + EPS)
    bx = beta_pre - mx * ax
    ax2, bx2 = ax.reshape(1, cin), bx.reshape(1, cin)

    h1, sums_h1 = pl.pallas_call(
        _h1_kernel, grid=(nb,),
        in_specs=[pl.BlockSpec((BR, cin), lambda i: (i, 0)),
                  pl.BlockSpec((1, cin), lambda i: (0, 0)),
                  pl.BlockSpec((1, cin), lambda i: (0, 0)),
                  pl.BlockSpec((cin, mid), lambda i: (0, 0)),
                  pl.BlockSpec((1, mid), lambda i: (0, 0))],
        out_specs=[pl.BlockSpec((BR, mid), lambda i: (i, 0)),
                   pl.BlockSpec((2, mid), lambda i: (0, 0))],
        out_shape=[jax.ShapeDtypeStruct((n, mid), F32),
                   jax.ShapeDtypeStruct((2, mid), F32)],
        scratch_shapes=[pltpu.VMEM((2, mid), F32)],
    )(x_feats, ax2, bx2, W1, b1.reshape(1, mid))
    m2 = sums_h1[0] / n
    a2 = gamma2 / jnp.sqrt(sums_h1[1] / n - m2 * m2 + EPS)
    b2f = beta2 - m2 * a2

    proj = pl.pallas_call(
        functools.partial(_proj_kernel, kk=kk), grid=(nb,),
        in_specs=[pl.BlockSpec((BR, mid), lambda i: (i, 0)),
                  pl.BlockSpec((1, mid), lambda i: (0, 0)),
                  pl.BlockSpec((1, mid), lambda i: (0, 0)),
                  pl.BlockSpec((kk, mid, mid), lambda i: (0, 0, 0))],
        out_specs=pl.BlockSpec((kk, BR, mid), lambda i: (0, i, 0)),
        out_shape=jax.ShapeDtypeStruct((kk, n, mid), F32),
    )(h1, a2.reshape(1, mid), b2f.reshape(1, mid), W2)
    p_flat = proj.reshape(kk * n, mid)

    acc_rows = (n // (NS * CHUNK) + 1) * NS * CHUNK
    n_pad = -(-n // (NS * 8)) * NS * 8
    mesh = plsc.VectorSubcoreMesh(core_axis_name="c", subcore_axis_name="s",
                                  num_cores=NC, num_subcores=NS)
    sc_fn = pl.kernel(
        functools.partial(_sc_body, ch=ch, acc_rows=acc_rows, n_pad=n_pad,
                          mid=mid),
        out_type=jax.ShapeDtypeStruct((NC, n_pad, mid), F32),
        mesh=mesh,
        compiler_params=pltpu.CompilerParams(use_tc_tiling_on_sc=False),
        scratch_types=[
            pltpu.VMEM((ch, CHUNK), jnp.int32),
            pltpu.VMEM((ch, CHUNK), jnp.int32),
            pltpu.VMEM((CHUNK, mid), F32),
            pltpu.VMEM((CHUNK, mid), F32),
            pltpu.VMEM_SHARED((acc_rows, mid), F32),
            pltpu.SemaphoreType.DMA,
        ],
    )
    partials = sc_fn(p_flat, g4, o4)

    sums_o = pl.pallas_call(
        _stats2_kernel, grid=(nb,),
        in_specs=[pl.BlockSpec((NC, BR, mid), lambda i: (0, i, 0)),
                  pl.BlockSpec((1, BR, mid), lambda i: (kc, i, 0))],
        out_specs=pl.BlockSpec((2, mid), lambda i: (0, 0)),
        out_shape=jax.ShapeDtypeStruct((2, mid), F32),
        scratch_shapes=[pltpu.VMEM((2, mid), F32)],
    )(partials, proj)
    m3 = sums_o[0] / n
    a3 = gamma3 / jnp.sqrt(sums_o[1] / n - m3 * m3 + EPS)
    b3f = beta3 - m3 * a3

    y = pl.pallas_call(
        _final_kernel, grid=(nb,),
        in_specs=[pl.BlockSpec((NC, BR, mid), lambda i: (0, i, 0)),
                  pl.BlockSpec((1, BR, mid), lambda i: (kc, i, 0)),
                  pl.BlockSpec((BR, cin), lambda i: (i, 0)),
                  pl.BlockSpec((1, mid), lambda i: (0, 0)),
                  pl.BlockSpec((1, mid), lambda i: (0, 0)),
                  pl.BlockSpec((mid, cout), lambda i: (0, 0)),
                  pl.BlockSpec((1, cout), lambda i: (0, 0)),
                  pl.BlockSpec((1, cin), lambda i: (0, 0)),
                  pl.BlockSpec((1, cin), lambda i: (0, 0))],
        out_specs=pl.BlockSpec((BR, cout), lambda i: (i, 0)),
        out_shape=jax.ShapeDtypeStruct((n, cout), F32),
    )(partials, proj, x_feats, a3.reshape(1, mid), b3f.reshape(1, mid),
      W3, b3.reshape(1, cout), ax2, bx2)
    return y


# speedup vs baseline: 24.4237x; 1.1585x over previous
import functools

import jax
import jax.numpy as jnp
from jax import lax
from jax.experimental import pallas as pl
from jax.experimental.pallas import tpu as pltpu
from jax.experimental.pallas import tpu_sc as plsc

F32 = jnp.float32
EPS = 1e-5
NC = 2
NS = 16
CHUNK = 128
ZROWS = 64
BR = 2000


def _stats_kernel(x_ref, o_ref, acc):
    i = pl.program_id(0)
    nb = pl.num_programs(0)
    xb = x_ref[...]
    upd = jnp.concatenate(
        [jnp.sum(xb, axis=0, keepdims=True),
         jnp.sum(xb * xb, axis=0, keepdims=True)], axis=0)
    acc[...] = jnp.where(i == 0, upd, acc[...] + upd)

    @pl.when(i == nb - 1)
    def _():
        o_ref[...] = acc[...]


def _h1_kernel(x_ref, a_ref, b_ref, w1_ref, b1_ref, h1_ref, s_ref, acc):
    i = pl.program_id(0)
    nb = pl.num_programs(0)
    xp = jnp.maximum(x_ref[...] * a_ref[...] + b_ref[...], 0.0)
    h1 = jnp.dot(xp, w1_ref[...], preferred_element_type=F32) + b1_ref[...]
    h1_ref[...] = h1
    upd = jnp.concatenate(
        [jnp.sum(h1, axis=0, keepdims=True),
         jnp.sum(h1 * h1, axis=0, keepdims=True)], axis=0)
    acc[...] = jnp.where(i == 0, upd, acc[...] + upd)

    @pl.when(i == nb - 1)
    def _():
        s_ref[...] = acc[...]


def _proj_kernel(h1_ref, a_ref, b_ref, w2_ref, p_ref, *, kk):
    h = jnp.maximum(h1_ref[...] * a_ref[...] + b_ref[...], 0.0)
    for k in range(kk):
        p_ref[k, :, :] = jnp.dot(h, w2_ref[k, :, :], preferred_element_type=F32)


def _stats2_kernel(pp_ref, o_ref, acc):
    i = pl.program_id(0)
    nb = pl.num_programs(0)
    s = pp_ref[0, :, :] + pp_ref[1, :, :]
    upd = jnp.concatenate(
        [jnp.sum(s, axis=0, keepdims=True),
         jnp.sum(s * s, axis=0, keepdims=True)], axis=0)
    acc[...] = jnp.where(i == 0, upd, acc[...] + upd)

    @pl.when(i == nb - 1)
    def _():
        o_ref[...] = acc[...]


def _final_kernel(pp_ref, x_ref, a3_ref, b3f_ref, w3_ref, bias3_ref,
                  ax_ref, bx_ref, y_ref):
    s = pp_ref[0, :, :] + pp_ref[1, :, :]
    h3 = jnp.maximum(s * a3_ref[...] + b3f_ref[...], 0.0)
    shortcut = jnp.maximum(x_ref[...] * ax_ref[...] + bx_ref[...], 0.0)
    y_ref[...] = (jnp.dot(h3, w3_ref[...], preferred_element_type=F32)
                  + bias3_ref[...] + shortcut)


def _sc_body(p_hbm, g_hbm, o_hbm, out_hbm, gix, oix, rows0, rows1, zbuf, acc,
             sem0, sem1, *, ch, acc_rows, n_pad, mid):
    c = lax.axis_index("c")
    s = lax.axis_index("s")

    def zrow(i, carry):
        for j in range(mid // 16):
            zbuf[i, pl.ds(j * 16, 16)] = jnp.zeros((16,), F32)
        return carry
    lax.fori_loop(0, ZROWS, zrow, 0)

    zps = acc_rows // NS
    def zcopy(r, carry):
        pltpu.sync_copy(zbuf, acc.at[pl.ds(s * zps + r * ZROWS, ZROWS)])
        return carry
    lax.fori_loop(0, zps // ZROWS, zcopy, 0)

    pltpu.sync_copy(g_hbm.at[c, s], gix)
    pltpu.sync_copy(o_hbm.at[c, s], oix)
    plsc.subcore_barrier()

    pltpu.async_copy(p_hbm.at[gix.at[0]], rows0, sem0)
    pltpu.async_copy(p_hbm.at[gix.at[1]], rows1, sem1)

    def body(t, carry):
        j = 2 * t
        pltpu.make_async_copy(p_hbm.at[gix.at[j]], rows0, sem0).wait()
        pltpu.sync_copy(rows0, acc.at[oix.at[j]], add=True)

        @pl.when(j + 2 < ch)
        def _():
            pltpu.async_copy(p_hbm.at[gix.at[j + 2]], rows0, sem0)

        pltpu.make_async_copy(p_hbm.at[gix.at[j + 1]], rows1, sem1).wait()
        pltpu.sync_copy(rows1, acc.at[oix.at[j + 1]], add=True)

        @pl.when(j + 3 < ch)
        def _():
            pltpu.async_copy(p_hbm.at[gix.at[j + 3]], rows1, sem1)
        return carry
    lax.fori_loop(0, ch // 2, body, 0)
    plsc.subcore_barrier()

    rps = n_pad // NS
    pltpu.sync_copy(acc.at[pl.ds(s * rps, rps)],
                    out_hbm.at[c, pl.ds(s * rps, rps)])


def kernel(x_feats, gamma_pre, beta_pre, W1, b1, gamma2, beta2, W2, b2,
           gamma3, beta3, W3, b3, edge_in, edge_out, cu_kernel):
    n, cin = x_feats.shape
    mid = W1.shape[1]
    cout = W3.shape[1]
    kk = W2.shape[0]
    e_total = edge_in.shape[0]
    nb = n // BR
    kc = kk // 2

    n_pad = -(-n // (NS * 8)) * NS * 8
    acc_rows = (n_pad // (NS * ZROWS) + 1) * NS * ZROWS
    cu = cu_kernel.astype(jnp.int32)
    pos = jnp.arange(e_total, dtype=jnp.int32)
    seg = jnp.zeros((e_total,), jnp.int32)
    for k in range(1, kk):
        seg = seg + (pos >= cu[k]).astype(jnp.int32)
    gidx = edge_in.astype(jnp.int32) + seg * n

    nw = NC * NS
    ch = -(-e_total // (nw * CHUNK))
    ch += ch % 2
    ep = nw * ch * CHUNK
    pad = ep - e_total
    padi = jnp.arange(pad, dtype=jnp.int32)
    gidx = jnp.concatenate([gidx, padi % jnp.int32(n)])
    oidx = jnp.concatenate([edge_out.astype(jnp.int32),
                            n_pad + padi % jnp.int32(acc_rows - n_pad)])
    g4 = gidx.reshape(NC, NS, ch, CHUNK)
    o4 = oidx.reshape(NC, NS, ch, CHUNK)

    sums_x = pl.pallas_call(
        _stats_kernel, grid=(nb,),
        in_specs=[pl.BlockSpec((BR, cin), lambda i: (i, 0))],
        out_specs=pl.BlockSpec((2, cin), lambda i: (0, 0)),
        out_shape=jax.ShapeDtypeStruct((2, cin), F32),
        scratch_shapes=[pltpu.VMEM((2, cin), F32)],
    )(x_feats)
    mx = sums_x[0] / n
    ax = gamma_pre / jnp.sqrt(sums_x[1] / n - mx * mx + EPS)
    bx = beta_pre - mx * ax
    ax2, bx2 = ax.reshape(1, cin), bx.reshape(1, cin)

    h1, sums_h1 = pl.pallas_call(
        _h1_kernel, grid=(nb,),
        in_specs=[pl.BlockSpec((BR, cin), lambda i: (i, 0)),
                  pl.BlockSpec((1, cin), lambda i: (0, 0)),
                  pl.BlockSpec((1, cin), lambda i: (0, 0)),
                  pl.BlockSpec((cin, mid), lambda i: (0, 0)),
                  pl.BlockSpec((1, mid), lambda i: (0, 0))],
        out_specs=[pl.BlockSpec((BR, mid), lambda i: (i, 0)),
                   pl.BlockSpec((2, mid), lambda i: (0, 0))],
        out_shape=[jax.ShapeDtypeStruct((n, mid), F32),
                   jax.ShapeDtypeStruct((2, mid), F32)],
        scratch_shapes=[pltpu.VMEM((2, mid), F32)],
    )(x_feats, ax2, bx2, W1, b1.reshape(1, mid))
    m2 = sums_h1[0] / n
    a2 = gamma2 / jnp.sqrt(sums_h1[1] / n - m2 * m2 + EPS)
    b2f = beta2 - m2 * a2

    proj = pl.pallas_call(
        functools.partial(_proj_kernel, kk=kk), grid=(nb,),
        in_specs=[pl.BlockSpec((BR, mid), lambda i: (i, 0)),
                  pl.BlockSpec((1, mid), lambda i: (0, 0)),
                  pl.BlockSpec((1, mid), lambda i: (0, 0)),
                  pl.BlockSpec((kk, mid, mid), lambda i: (0, 0, 0))],
        out_specs=pl.BlockSpec((kk, BR, mid), lambda i: (0, i, 0)),
        out_shape=jax.ShapeDtypeStruct((kk, n, mid), F32),
    )(h1, a2.reshape(1, mid), b2f.reshape(1, mid), W2)
    p_flat = proj.reshape(kk * n, mid)

    mesh = plsc.VectorSubcoreMesh(core_axis_name="c", subcore_axis_name="s",
                                  num_cores=NC, num_subcores=NS)
    sc_fn = pl.kernel(
        functools.partial(_sc_body, ch=ch, acc_rows=acc_rows, n_pad=n_pad,
                          mid=mid),
        out_type=jax.ShapeDtypeStruct((NC, n_pad, mid), F32),
        mesh=mesh,
        compiler_params=pltpu.CompilerParams(use_tc_tiling_on_sc=False),
        scratch_types=[
            pltpu.VMEM((ch, CHUNK), jnp.int32),
            pltpu.VMEM((ch, CHUNK), jnp.int32),
            pltpu.VMEM((CHUNK, mid), F32),
            pltpu.VMEM((CHUNK, mid), F32),
            pltpu.VMEM((ZROWS, mid), F32),
            pltpu.VMEM_SHARED((acc_rows, mid), F32),
            pltpu.SemaphoreType.DMA,
            pltpu.SemaphoreType.DMA,
        ],
    )
    partials = sc_fn(p_flat, g4, o4)

    sums_o = pl.pallas_call(
        _stats2_kernel, grid=(nb,),
        in_specs=[pl.BlockSpec((NC, BR, mid), lambda i: (0, i, 0))],
        out_specs=pl.BlockSpec((2, mid), lambda i: (0, 0)),
        out_shape=jax.ShapeDtypeStruct((2, mid), F32),
        scratch_shapes=[pltpu.VMEM((2, mid), F32)],
    )(partials)
    m3 = sums_o[0] / n
    a3 = gamma3 / jnp.sqrt(sums_o[1] / n - m3 * m3 + EPS)
    b3f = beta3 - m3 * a3

    y = pl.pallas_call(
        _final_kernel, grid=(nb,),
        in_specs=[pl.BlockSpec((NC, BR, mid), lambda i: (0, i, 0)),
                  pl.BlockSpec((BR, cin), lambda i: (i, 0)),
                  pl.BlockSpec((1, mid), lambda i: (0, 0)),
                  pl.BlockSpec((1, mid), lambda i: (0, 0)),
                  pl.BlockSpec((mid, cout), lambda i: (0, 0)),
                  pl.BlockSpec((1, cout), lambda i: (0, 0)),
                  pl.BlockSpec((1, cin), lambda i: (0, 0)),
                  pl.BlockSpec((1, cin), lambda i: (0, 0))],
        out_specs=pl.BlockSpec((BR, cout), lambda i: (i, 0)),
        out_shape=jax.ShapeDtypeStruct((n, cout), F32),
    )(partials, x_feats, a3.reshape(1, mid), b3f.reshape(1, mid),
      W3, b3.reshape(1, cout), ax2, bx2)
    return y


# speedup vs baseline: 53.9698x; 2.2097x over previous
import functools

import jax
import jax.numpy as jnp
from jax import lax
from jax.experimental import pallas as pl
from jax.experimental.pallas import tpu as pltpu
from jax.experimental.pallas import tpu_sc as plsc

F32 = jnp.float32
EPS = 1e-5
NC = 2
NS = 16
CHUNK = 128
ZROWS = 64
PK = 4


def _stats_kernel(x_ref, o_ref, acc):
    i = pl.program_id(0)
    nb = pl.num_programs(0)
    xb = x_ref[...]
    upd = jnp.concatenate(
        [jnp.sum(xb, axis=0, keepdims=True),
         jnp.sum(xb * xb, axis=0, keepdims=True)], axis=0)
    acc[...] = jnp.where(i == 0, upd, acc[...] + upd)

    @pl.when(i == nb - 1)
    def _():
        o_ref[...] = acc[...]


def _h1_kernel(x4_ref, a_ref, b_ref, w1_ref, b1_ref, lim_ref, h1_ref, s_ref,
               acc, *, br):
    i = pl.program_id(0)
    nb = pl.num_programs(0)
    h1 = b1_ref[...]
    for q in range(PK):
        xq = jnp.maximum(x4_ref[q, :, :] * a_ref[...] + b_ref[...], 0.0)
        h1 = h1 + jnp.dot(xq, w1_ref[q, :, :], preferred_element_type=F32)
    row = i * br + lax.broadcasted_iota(jnp.int32, h1.shape, 0)
    h1 = jnp.where(row < lim_ref[...], h1, 0.0)
    h1_ref[...] = h1
    upd = jnp.concatenate(
        [jnp.sum(h1, axis=0, keepdims=True),
         jnp.sum(h1 * h1, axis=0, keepdims=True)], axis=0)
    acc[...] = jnp.where(i == 0, upd, acc[...] + upd)

    @pl.when(i == nb - 1)
    def _():
        s_ref[...] = acc[...]


def _proj_kernel(h1_ref, a_ref, b_ref, w2_ref, p_ref, *, kk):
    h = jnp.maximum(h1_ref[...] * a_ref[...] + b_ref[...], 0.0)
    for k in range(kk):
        p_ref[k, :, :] = jnp.dot(h, w2_ref[k, :, :], preferred_element_type=F32)


def _stats2_kernel(pp_ref, o_ref, acc):
    i = pl.program_id(0)
    nb = pl.num_programs(0)
    s = pp_ref[0, :, :] + pp_ref[1, :, :]
    upd = jnp.concatenate(
        [jnp.sum(s, axis=0, keepdims=True),
         jnp.sum(s * s, axis=0, keepdims=True)], axis=0)
    acc[...] = jnp.where(i == 0, upd, acc[...] + upd)

    @pl.when(i == nb - 1)
    def _():
        o_ref[...] = acc[...]


def _final_kernel(pp_ref, x4_ref, a3_ref, b3f_ref, w3_ref, bias3_ref,
                  ax_ref, bx_ref, y4_ref):
    s = pp_ref[0, :, :] + pp_ref[1, :, :]
    h3 = jnp.maximum(s * a3_ref[...] + b3f_ref[...], 0.0)
    ycat = jnp.dot(h3, w3_ref[...], preferred_element_type=F32)
    for q in range(PK):
        shortcut = jnp.maximum(x4_ref[q, :, :] * ax_ref[...] + bx_ref[...],
                               0.0)
        y4_ref[q, :, :] = (ycat[:, q * 128:(q + 1) * 128] + bias3_ref[...]
                           + shortcut)


def _sc_body(p_hbm, g_hbm, o_hbm, out_hbm, gix, oix, rows0, rows1, zbuf, acc,
             sem0, sem1, *, ch, acc_rows, n_pad, mid):
    c = lax.axis_index("c")
    s = lax.axis_index("s")

    def zrow(i, carry):
        for j in range(mid // 16):
            zbuf[i, pl.ds(j * 16, 16)] = jnp.zeros((16,), F32)
        return carry
    lax.fori_loop(0, ZROWS, zrow, 0)

    zps = acc_rows // NS
    def zcopy(r, carry):
        pltpu.sync_copy(zbuf, acc.at[pl.ds(s * zps + r * ZROWS, ZROWS)])
        return carry
    lax.fori_loop(0, zps // ZROWS, zcopy, 0)

    pltpu.sync_copy(g_hbm.at[c, s], gix)
    pltpu.sync_copy(o_hbm.at[c, s], oix)
    plsc.subcore_barrier()

    pltpu.async_copy(p_hbm.at[gix.at[0]], rows0, sem0)
    pltpu.async_copy(p_hbm.at[gix.at[1]], rows1, sem1)

    def body(t, carry):
        j = 2 * t
        pltpu.make_async_copy(p_hbm.at[gix.at[j]], rows0, sem0).wait()
        pltpu.sync_copy(rows0, acc.at[oix.at[j]], add=True)

        @pl.when(j + 2 < ch)
        def _():
            pltpu.async_copy(p_hbm.at[gix.at[j + 2]], rows0, sem0)

        pltpu.make_async_copy(p_hbm.at[gix.at[j + 1]], rows1, sem1).wait()
        pltpu.sync_copy(rows1, acc.at[oix.at[j + 1]], add=True)

        @pl.when(j + 3 < ch)
        def _():
            pltpu.async_copy(p_hbm.at[gix.at[j + 3]], rows1, sem1)
        return carry
    lax.fori_loop(0, ch // 2, body, 0)
    plsc.subcore_barrier()

    rps = n_pad // NS
    pltpu.sync_copy(acc.at[pl.ds(s * rps, rps)],
                    out_hbm.at[c, pl.ds(s * rps, rps)])


def kernel(x_feats, gamma_pre, beta_pre, W1, b1, gamma2, beta2, W2, b2,
           gamma3, beta3, W3, b3, edge_in, edge_out, cu_kernel):
    n, cin = x_feats.shape
    mid = W1.shape[1]
    cout = W3.shape[1]
    kk = W2.shape[0]
    e_total = edge_in.shape[0]
    brp = 3128
    nqp = -(-(n // PK) // brp) * brp
    n_pad = nqp * PK
    eye4 = jnp.eye(PK, dtype=F32)

    n_pad16 = -(-n_pad // (NS * 8)) * NS * 8
    acc_rows = (n_pad16 // (NS * ZROWS) + 1) * NS * ZROWS
    cu = cu_kernel.astype(jnp.int32)
    pos = jnp.arange(e_total, dtype=jnp.int32)
    seg = jnp.zeros((e_total,), jnp.int32)
    for k in range(1, kk):
        seg = seg + (pos >= cu[k]).astype(jnp.int32)
    ein = edge_in.astype(jnp.int32)
    eout = edge_out.astype(jnp.int32)
    pin = PK * (ein % nqp) + ein // nqp
    pout = PK * (eout % nqp) + eout // nqp
    gidx = pin + seg * n_pad

    nw = NC * NS
    ch = -(-e_total // (nw * CHUNK))
    ch += ch % 2
    ep = nw * ch * CHUNK
    pad = ep - e_total
    padi = jnp.arange(pad, dtype=jnp.int32)
    gidx = jnp.concatenate([gidx, padi % jnp.int32(n)])
    oidx = jnp.concatenate([pout, n_pad + padi % jnp.int32(acc_rows - n_pad)])
    g4 = gidx.reshape(NC, NS, ch, CHUNK)
    o4 = oidx.reshape(NC, NS, ch, CHUNK)

    x_pad = jnp.pad(x_feats, ((0, n_pad - n), (0, 0)))
    x4 = x_pad.reshape(PK, nqp, cin)

    brx = 3128
    nbx = n_pad // brx
    sums_x = pl.pallas_call(
        _stats_kernel, grid=(nbx,),
        in_specs=[pl.BlockSpec((brx, cin), lambda i: (i, 0))],
        out_specs=pl.BlockSpec((2, cin), lambda i: (0, 0)),
        out_shape=jax.ShapeDtypeStruct((2, cin), F32),
        scratch_shapes=[pltpu.VMEM((2, cin), F32)],
    )(x_pad)
    mx = sums_x[0] / n
    ax = gamma_pre / jnp.sqrt(sums_x[1] / n - mx * mx + EPS)
    bx = beta_pre - mx * ax
    ax2, bx2 = ax.reshape(1, cin), bx.reshape(1, cin)

    br1 = 3128
    nb1 = nqp // br1
    lims = tuple(min(nqp, max(0, n - u * nqp)) for u in range(PK))
    w1bd = (W1[None, :, None, :] * eye4[:, None, :, None]).reshape(
        PK, cin, PK * mid)
    limvec = jnp.repeat(jnp.array(lims, jnp.int32), mid).reshape(1, PK * mid)
    h1p, sums_h1c = pl.pallas_call(
        functools.partial(_h1_kernel, br=br1),
        grid=(nb1,),
        in_specs=[pl.BlockSpec((PK, br1, cin), lambda i: (0, i, 0)),
                  pl.BlockSpec((1, cin), lambda i: (0, 0)),
                  pl.BlockSpec((1, cin), lambda i: (0, 0)),
                  pl.BlockSpec((PK, cin, PK * mid), lambda i: (0, 0, 0)),
                  pl.BlockSpec((1, PK * mid), lambda i: (0, 0)),
                  pl.BlockSpec((1, PK * mid), lambda i: (0, 0))],
        out_specs=[pl.BlockSpec((br1, PK * mid), lambda i: (i, 0)),
                   pl.BlockSpec((2, PK * mid), lambda i: (0, 0))],
        out_shape=[jax.ShapeDtypeStruct((nqp, PK * mid), F32),
                   jax.ShapeDtypeStruct((2, PK * mid), F32)],
        scratch_shapes=[pltpu.VMEM((2, PK * mid), F32)],
    )(x4, ax2, bx2, w1bd, jnp.tile(b1, PK).reshape(1, PK * mid), limvec)
    sums_h1 = sums_h1c.reshape(2, PK, mid).sum(axis=1)
    m2 = sums_h1[0] / n
    a2 = gamma2 / jnp.sqrt(sums_h1[1] / n - m2 * m2 + EPS)
    b2f = beta2 - m2 * a2

    br2 = 3128
    nb2 = nqp // br2
    w2bd = W2[:, None, :, None, :] * eye4[None, :, None, :, None]
    w2bd = w2bd.reshape(kk, PK * mid, PK * mid)
    proj = pl.pallas_call(
        functools.partial(_proj_kernel, kk=kk), grid=(nb2,),
        in_specs=[pl.BlockSpec((br2, PK * mid), lambda i: (i, 0)),
                  pl.BlockSpec((1, PK * mid), lambda i: (0, 0)),
                  pl.BlockSpec((1, PK * mid), lambda i: (0, 0)),
                  pl.BlockSpec((kk, PK * mid, PK * mid),
                               lambda i: (0, 0, 0))],
        out_specs=pl.BlockSpec((kk, br2, PK * mid), lambda i: (0, i, 0)),
        out_shape=jax.ShapeDtypeStruct((kk, nqp, PK * mid), F32),
    )(h1p, jnp.tile(a2, PK).reshape(1, PK * mid),
      jnp.tile(b2f, PK).reshape(1, PK * mid), w2bd)
    p_flat = proj.reshape(kk * n_pad, mid)

    mesh = plsc.VectorSubcoreMesh(core_axis_name="c", subcore_axis_name="s",
                                  num_cores=NC, num_subcores=NS)
    sc_fn = pl.kernel(
        functools.partial(_sc_body, ch=ch, acc_rows=acc_rows, n_pad=n_pad,
                          mid=mid),
        out_type=jax.ShapeDtypeStruct((NC, n_pad, mid), F32),
        mesh=mesh,
        compiler_params=pltpu.CompilerParams(use_tc_tiling_on_sc=False),
        scratch_types=[
            pltpu.VMEM((ch, CHUNK), jnp.int32),
            pltpu.VMEM((ch, CHUNK), jnp.int32),
            pltpu.VMEM((CHUNK, mid), F32),
            pltpu.VMEM((CHUNK, mid), F32),
            pltpu.VMEM((ZROWS, mid), F32),
            pltpu.VMEM_SHARED((acc_rows, mid), F32),
            pltpu.SemaphoreType.DMA,
            pltpu.SemaphoreType.DMA,
        ],
    )
    partials = sc_fn(p_flat, g4, o4).reshape(NC, nqp, PK * mid)

    sums_oc = pl.pallas_call(
        _stats2_kernel, grid=(nb2,),
        in_specs=[pl.BlockSpec((NC, br2, PK * mid), lambda i: (0, i, 0))],
        out_specs=pl.BlockSpec((2, PK * mid), lambda i: (0, 0)),
        out_shape=jax.ShapeDtypeStruct((2, PK * mid), F32),
        scratch_shapes=[pltpu.VMEM((2, PK * mid), F32)],
    )(partials)
    sums_o = sums_oc.reshape(2, PK, mid).sum(axis=1)
    m3 = sums_o[0] / n
    a3 = gamma3 / jnp.sqrt(sums_o[1] / n - m3 * m3 + EPS)
    b3f = beta3 - m3 * a3

    w3k = (eye4[:, None, :, None] * W3[None, :, None, :]).reshape(
        PK * mid, PK * cout)
    brf = 3128
    nbf = nqp // brf
    y4 = pl.pallas_call(
        _final_kernel, grid=(nbf,),
        in_specs=[pl.BlockSpec((NC, brf, PK * mid), lambda i: (0, i, 0)),
                  pl.BlockSpec((PK, brf, cin), lambda i: (0, i, 0)),
                  pl.BlockSpec((1, PK * mid), lambda i: (0, 0)),
                  pl.BlockSpec((1, PK * mid), lambda i: (0, 0)),
                  pl.BlockSpec((PK * mid, PK * cout), lambda i: (0, 0)),
                  pl.BlockSpec((1, cout), lambda i: (0, 0)),
                  pl.BlockSpec((1, cin), lambda i: (0, 0)),
                  pl.BlockSpec((1, cin), lambda i: (0, 0))],
        out_specs=pl.BlockSpec((PK, brf, cout), lambda i: (0, i, 0)),
        out_shape=jax.ShapeDtypeStruct((PK, nqp, cout), F32),
    )(partials, x4, jnp.tile(a3, PK).reshape(1, PK * mid),
      jnp.tile(b3f, PK).reshape(1, PK * mid), w3k, b3.reshape(1, cout),
      ax2, bx2)
    return y4.reshape(n_pad, cout)[:n]
